# baseline (speedup 1.0000x reference)
import jax
import jax.numpy as jnp
from jax.experimental import pallas as pl
from jax.experimental.pallas import tpu as pltpu


_R = 8000
_WINDOW = 16


_NSEM = 8


def _body(w_ref, o_ref, buf, sems):
    buf[...] = jnp.broadcast_to(w_ref[...], buf.shape)
    n = o_ref.shape[0] // _R
    for k in range(n):
        pltpu.make_async_copy(
            buf, o_ref.at[pl.ds(k * _R, _R)], sems.at[k % _NSEM]).start()
        if k >= _WINDOW:
            j = k - _WINDOW
            pltpu.make_async_copy(
                buf, o_ref.at[pl.ds(j * _R, _R)], sems.at[j % _NSEM]).wait()
    for k in range(max(n - _WINDOW, 0), n):
        pltpu.make_async_copy(
            buf, o_ref.at[pl.ds(k * _R, _R)], sems.at[k % _NSEM]).wait()


def kernel(edge_index, W):
    E = edge_index.shape[1]
    D = W.shape[1]
    return pl.pallas_call(
        _body,
        in_specs=[pl.BlockSpec(memory_space=pltpu.MemorySpace.VMEM)],
        out_specs=pl.BlockSpec(memory_space=pltpu.MemorySpace.HBM),
        out_shape=jax.ShapeDtypeStruct((E, D), jnp.float32),
        scratch_shapes=[
            pltpu.MemorySpace.VMEM((_R, D), jnp.float32),
            pltpu.SemaphoreType.DMA((_NSEM,)),
        ],
    )(W)


# speedup vs baseline: 6.3961x; 6.3961x over previous
import jax
import jax.numpy as jnp
from jax.experimental import pallas as pl


_BLOCK_C = 16000


def _broadcast_body(w_ref, o_ref):
    o_ref[...] = jnp.broadcast_to(w_ref[...], o_ref.shape)


def kernel(edge_index, W):
    E = edge_index.shape[1]
    D = W.shape[1]
    w_col = W.reshape(D, 1)
    out_t = pl.pallas_call(
        _broadcast_body,
        grid=(E // _BLOCK_C,),
        in_specs=[pl.BlockSpec((D, 1), lambda i: (0, 0))],
        out_specs=pl.BlockSpec((D, _BLOCK_C), lambda i: (0, i)),
        out_shape=jax.ShapeDtypeStruct((D, E), jnp.float32),
    )(w_col)
    return out_t.T
